# baseline (speedup 1.0000x reference)
import functools

import jax
import jax.numpy as jnp
from jax import lax
from jax.experimental import pallas as pl
from jax.experimental.pallas import tpu as pltpu
from jax.experimental.pallas import tpu_sc as plsc

B = 32
HB = B // 2
N = 4096
L = 512
CHUNK = 4096
NCH = N // CHUNK


def _argmax_body(x_ref, idx_ref):
    x = x_ref[0]
    m = jnp.max(x, axis=-1, keepdims=True)
    iota = lax.broadcasted_iota(jnp.int32, (CHUNK, L), 1).astype(jnp.float32)
    idxf = jnp.min(jnp.where(x == m, iota, float(L)), axis=-1, keepdims=True)
    idx_ref[...] = idxf.astype(jnp.int32).reshape(1, 1, CHUNK // 128, 128)


@functools.partial(jax.jit, static_argnums=1)
def _argmax_tc_half(x, half):
    off = half * HB
    return pl.pallas_call(
        _argmax_body,
        grid=(HB, NCH),
        in_specs=[pl.BlockSpec((1, CHUNK, L), lambda b, c: (b + off, c, 0))],
        out_specs=pl.BlockSpec(
            (1, 1, CHUNK // 128, 128), lambda b, c: (b, c, 0, 0)
        ),
        out_shape=jax.ShapeDtypeStruct((HB, NCH, CHUNK // 128, 128), jnp.int32),
    )(x)


_mesh = plsc.VectorSubcoreMesh(core_axis_name="c", subcore_axis_name="s")


@functools.partial(
    pl.kernel,
    mesh=_mesh,
    out_type=jax.ShapeDtypeStruct((HB, L), jnp.float32),
    scratch_types=[
        pltpu.VMEM((N,), jnp.int32),
        pltpu.VMEM((L,), jnp.float32),
    ],
    compiler_params=pltpu.CompilerParams(needs_layout_passes=False),
)
def _hist_sc_half(idx_hbm, out_hbm, idx_v, hist_v):
    row = lax.axis_index("s")
    pltpu.sync_copy(idx_hbm.at[row], idx_v)
    zeros = jnp.zeros((16,), jnp.float32)
    for i in range(L // 16):
        hist_v[pl.ds(i * 16, 16)] = zeros
    ones = jnp.ones((16,), jnp.float32)

    def body(i, carry):
        iv = idx_v[pl.ds(i * 16, 16)]
        plsc.addupdate_scatter(hist_v, [iv], ones)
        return carry

    lax.fori_loop(0, N // 16, body, 0)
    pltpu.sync_copy(hist_v, out_hbm.at[row])


def kernel(x):
    idx0 = _argmax_tc_half(x, 0).reshape(HB, N)
    h0 = _hist_sc_half(idx0)
    idx1 = _argmax_tc_half(x, 1).reshape(HB, N)
    h1 = _hist_sc_half(idx1)
    return jnp.concatenate([h0, h1], axis=0)


# speedup vs baseline: 1.0001x; 1.0001x over previous
import functools

import jax
import jax.numpy as jnp
from jax import lax
from jax.experimental import pallas as pl
from jax.experimental.pallas import tpu as pltpu
from jax.experimental.pallas import tpu_sc as plsc

B = 32
HB = B // 2
N = 4096
L = 512
CHUNK = 4096
NCH = N // CHUNK


def _argmax_body(x_ref, idx_ref):
    x = x_ref[0]
    m = jnp.max(x, axis=-1, keepdims=True)
    iota = lax.broadcasted_iota(jnp.int32, (CHUNK, L), 1).astype(jnp.float32)
    idxf = jnp.min(jnp.where(x == m, iota, float(L)), axis=-1, keepdims=True)
    idx_ref[...] = idxf.astype(jnp.int32).reshape(1, 1, CHUNK // 128, 128)


@functools.partial(jax.jit, static_argnums=1)
def _argmax_tc_half(x, half):
    off = half * HB
    return pl.pallas_call(
        _argmax_body,
        grid=(HB, NCH),
        in_specs=[pl.BlockSpec((1, CHUNK, L), lambda b, c: (b + off, c, 0))],
        out_specs=pl.BlockSpec(
            (1, 1, CHUNK // 128, 128), lambda b, c: (b, c, 0, 0)
        ),
        out_shape=jax.ShapeDtypeStruct((HB, NCH, CHUNK // 128, 128), jnp.int32),
        compiler_params=pltpu.CompilerParams(
            dimension_semantics=("parallel", "parallel")
        ),
    )(x)


_mesh = plsc.VectorSubcoreMesh(core_axis_name="c", subcore_axis_name="s")


@functools.partial(
    pl.kernel,
    mesh=_mesh,
    out_type=jax.ShapeDtypeStruct((HB, L), jnp.float32),
    scratch_types=[
        pltpu.VMEM((N,), jnp.int32),
        pltpu.VMEM((L,), jnp.float32),
    ],
    compiler_params=pltpu.CompilerParams(needs_layout_passes=False),
)
def _hist_sc_half(idx_hbm, out_hbm, idx_v, hist_v):
    row = lax.axis_index("s")
    pltpu.sync_copy(idx_hbm.at[row], idx_v)
    zeros = jnp.zeros((16,), jnp.float32)
    for i in range(L // 16):
        hist_v[pl.ds(i * 16, 16)] = zeros
    ones = jnp.ones((16,), jnp.float32)

    def body(i, carry):
        iv = idx_v[pl.ds(i * 16, 16)]
        plsc.addupdate_scatter(hist_v, [iv], ones)
        return carry

    lax.fori_loop(0, N // 16, body, 0)
    pltpu.sync_copy(hist_v, out_hbm.at[row])


def kernel(x):
    idx0 = _argmax_tc_half(x, 0).reshape(HB, N)
    h0 = _hist_sc_half(idx0)
    idx1 = _argmax_tc_half(x, 1).reshape(HB, N)
    h1 = _hist_sc_half(idx1)
    return jnp.concatenate([h0, h1], axis=0)
